# baseline (speedup 1.0000x reference)
import functools

import functools

import jax
import jax.numpy as jnp
from jax.experimental import pallas as pl
from jax.experimental.pallas import tpu as pltpu
from jax.experimental.pallas import tpu_sc as plsc

N = 10000
D = 128
HID = 128
OUT = 256
RATIO = 0.5
R = 0.2
MAX_NB = 32
M = int(N * RATIO)

SUB = 8
LANES = -(-N // (SUB * 128)) * 128
TOTAL = SUB * LANES


def _fps_body(posr_ref, xg_ref, yg_ref, zg_ref, idx_ref, cen_ref):
    X = xg_ref[:, :]
    Y = yg_ref[:, :]
    Z = zg_ref[:, :]
    sub_i = jax.lax.broadcasted_iota(jnp.int32, (SUB, LANES), 0)
    lane_i = jax.lax.broadcasted_iota(jnp.int32, (SUB, LANES), 1)
    iota = sub_i * LANES + lane_i
    valid = iota < N
    big = jnp.int32(N)
    dd0 = jnp.where(valid, jnp.inf, -jnp.inf).astype(jnp.float32)

    idx_ref[pl.ds(0, 1), :] = jnp.zeros((1, 1), jnp.int32)
    cen_ref[pl.ds(0, 1), :] = posr_ref[pl.ds(0, 1), :]

    def body(i, carry):
        dd, last = carry
        rowp = posr_ref[pl.ds(last, 1), :]
        bx = jnp.broadcast_to(rowp[0:1, 0:1], (SUB, LANES))
        by = jnp.broadcast_to(rowp[0:1, 1:2], (SUB, LANES))
        bz = jnp.broadcast_to(rowp[0:1, 2:3], (SUB, LANES))
        dx = X - bx
        dy = Y - by
        dz = Z - bz
        d = dx * dx + dy * dy + dz * dz
        dd = jnp.minimum(dd, d)
        mx = jnp.max(dd)
        nxt = jnp.min(jnp.where(dd == mx, iota, big)).astype(jnp.int32)
        idx_ref[pl.ds(i, 1), :] = jnp.full((1, 1), 0, jnp.int32) + nxt
        cen_ref[pl.ds(i, 1), :] = posr_ref[pl.ds(nxt, 1), :]
        return (dd, nxt)

    jax.lax.fori_loop(1, M, body, (dd0, jnp.int32(0)))


def _fps_pallas(pos):
    pad = jnp.zeros((TOTAL - N,), jnp.float32)
    xg = jnp.concatenate([pos[:, 0], pad]).reshape(SUB, LANES)
    yg = jnp.concatenate([pos[:, 1], pad]).reshape(SUB, LANES)
    zg = jnp.concatenate([pos[:, 2], pad]).reshape(SUB, LANES)
    idx2, cen = pl.pallas_call(
        _fps_body,
        out_shape=(
            jax.ShapeDtypeStruct((M, 1), jnp.int32),
            jax.ShapeDtypeStruct((M, 3), jnp.float32),
        ),
    )(pos, xg, yg, zg)
    return idx2[:, 0], cen


QBITS = 17
JBITS = 14
JMASK = (1 << JBITS) - 1
KSCALE = float((2 ** QBITS - 1) / (R * R))
SENT = 2 ** 31 - 1
PPAD = 10240
MPAD = 5120
CBLK = 256
PBLK = 2560


def _keys_body(cen_ref, posT_ref, pn2_ref, keys_ref):
    c = cen_ref[:, :]
    pT = posT_ref[:, :]
    pn2 = pn2_ref[:, :]
    cn2 = jnp.sum(c * c, axis=1, keepdims=True)
    dot = jnp.dot(c, pT, preferred_element_type=jnp.float32)
    d2 = cn2 + pn2 - 2.0 * dot
    q = jnp.maximum(jnp.int32(0), (d2 * KSCALE).astype(jnp.int32))
    j = jax.lax.broadcasted_iota(jnp.int32, (CBLK, PBLK), 1)
    base = pl.program_id(1) * PBLK
    key = jnp.where(d2 <= R * R, (q << JBITS) | (j + base), jnp.int32(SENT))
    keys_ref[:, :] = key


def _keys_pallas(centers, pos):
    cenp = jnp.concatenate(
        [centers, jnp.full((MPAD - M, 3), 100.0, jnp.float32)], axis=0)
    posT = jnp.concatenate(
        [pos.T, jnp.full((3, PPAD - N), 50.0, jnp.float32)], axis=1)
    pn2 = jnp.sum(posT * posT, axis=0, keepdims=True)
    keys = pl.pallas_call(
        _keys_body,
        grid=(MPAD // CBLK, PPAD // PBLK),
        in_specs=[
            pl.BlockSpec((CBLK, 3), lambda i, j: (i, 0)),
            pl.BlockSpec((3, PBLK), lambda i, j: (0, j)),
            pl.BlockSpec((1, PBLK), lambda i, j: (0, j)),
        ],
        out_specs=pl.BlockSpec((CBLK, PBLK), lambda i, j: (i, j)),
        out_shape=jax.ShapeDtypeStruct((MPAD, PPAD), jnp.int32),
    )(cenp, posT, pn2)
    return keys


SC_NW = 32
SC_ROWS = MPAD // SC_NW
CAND = 1024
NVMAX = CAND // 16


def _sc_select_body(keys_hbm, nbr_hbm, cnt_hbm, rowbuf, cand, nbr_st, cnt_st):
    c = jax.lax.axis_index("c")
    s = jax.lax.axis_index("s")
    wid = s * 2 + c
    base = wid * SC_ROWS
    sentv = jnp.full((16,), SENT, jnp.int32)
    zeros16 = jnp.zeros((16,), jnp.int32)
    lane = jax.lax.iota(jnp.int32, 16)

    def per_row(r, _):
        pltpu.sync_copy(keys_hbm.at[base + r], rowbuf)

        def pf(g, carry):
            cand[pl.ds(g * 16, 16)] = sentv
            return carry

        jax.lax.fori_loop(0, NVMAX, pf, 0)

        def fb(g, ptr):
            k16 = rowbuf[pl.ds(g * 16, 16)]
            m = k16 < jnp.int32(SENT)
            ones = jnp.where(m, jnp.int32(1), jnp.int32(0))
            pos = plsc.cumsum(ones) + ptr - 1
            pos = jnp.minimum(pos, jnp.int32(CAND - 1))
            plsc.store_scatter(cand, [pos], k16, mask=m)
            return ptr + plsc.all_reduce_population_count(m)

        ptr = jax.lax.fori_loop(0, PPAD // 16, fb, zeros16)
        cnt = jnp.minimum(jnp.max(ptr), jnp.int32(CAND))
        nv = (cnt + 15) // 16

        def ib(i, carry):
            sel0, sel1 = carry

            def mb(v, acc):
                return jnp.minimum(acc, cand[pl.ds(v * 16, 16)])

            mv = jax.lax.fori_loop(0, nv, mb, sentv)
            mns = jnp.broadcast_to(jnp.min(mv), (16,))
            sel0 = jnp.where(lane == i, mns, sel0)
            sel1 = jnp.where(lane == (i - 16), mns, sel1)

            def rb(v, carry2):
                cv = cand[pl.ds(v * 16, 16)]
                cand[pl.ds(v * 16, 16)] = jnp.where(cv == mns, sentv, cv)
                return carry2

            jax.lax.fori_loop(0, nv, rb, 0)
            return (sel0, sel1)

        sel0, sel1 = jax.lax.fori_loop(0, 32, ib, (sentv, sentv))
        v0 = sel0 != jnp.int32(SENT)
        v1 = sel1 != jnp.int32(SENT)
        jm = jnp.full((16,), JMASK, jnp.int32)
        nbr_st[pl.ds(r * 32, 16)] = jnp.where(v0, sel0 & jm, zeros16)
        nbr_st[pl.ds(r * 32 + 16, 16)] = jnp.where(v1, sel1 & jm, zeros16)
        nsel = (plsc.all_reduce_population_count(v0)
                + plsc.all_reduce_population_count(v1))
        cnt_st[pl.ds(r * 16, 16)] = nsel
        return _

    jax.lax.fori_loop(0, SC_ROWS, per_row, 0)
    pltpu.sync_copy(nbr_st, nbr_hbm.at[pl.ds(base * 32, SC_ROWS * 32)])
    pltpu.sync_copy(cnt_st, cnt_hbm.at[pl.ds(base * 16, SC_ROWS * 16)])


def _sc_select(keys):
    mesh = plsc.VectorSubcoreMesh(core_axis_name="c", subcore_axis_name="s")
    fn = functools.partial(
        pl.kernel,
        mesh=mesh,
        out_type=(
            jax.ShapeDtypeStruct((MPAD * 32,), jnp.int32),
            jax.ShapeDtypeStruct((MPAD * 16,), jnp.int32),
        ),
        scratch_types=[
            pltpu.VMEM((PPAD,), jnp.int32),
            pltpu.VMEM((CAND,), jnp.int32),
            pltpu.VMEM((SC_ROWS * 32,), jnp.int32),
            pltpu.VMEM((SC_ROWS * 16,), jnp.int32),
        ],
    )(_sc_select_body)
    nbr_flat, cnt_flat = fn(keys)
    nbr = nbr_flat.reshape(MPAD, 32)[:M]
    cnt = cnt_flat.reshape(MPAD, 16)[:M, 0]
    valid = jnp.arange(32, dtype=jnp.int32)[None, :] < cnt[:, None]
    return nbr, valid


def _select_emulated(keys):
    skey = jax.lax.top_k(-keys[:M], MAX_NB)[0]
    skey = -skey
    valid = skey != SENT
    nbr = jnp.where(valid, skey & JMASK, 0)
    return nbr, valid


def _radius_v0(pos, centers):
    pn2 = jnp.sum(pos ** 2, axis=-1)
    cn2 = jnp.sum(centers ** 2, axis=-1)
    d2 = cn2[:, None] + pn2[None, :] - 2.0 * (centers @ pos.T)
    neg = jnp.where(d2 <= R * R, -d2, -jnp.inf)
    vals, nbr = jax.lax.top_k(neg, MAX_NB)
    valid = jnp.isfinite(vals)
    return nbr.astype(jnp.int32), valid


def kernel(x, pos, W_msg, b_msg, W_pos, b_pos, W_upd, b_upd, batch):
    idx, centers = _fps_pallas(pos)
    keys = _keys_pallas(centers, pos)
    nbr, valid = _select_emulated(keys)
    row = jnp.broadcast_to(jnp.arange(M, dtype=jnp.int32)[:, None], (M, MAX_NB)).reshape(-1)
    col = nbr.reshape(-1)
    vmask = valid.reshape(-1).astype(jnp.float32)
    x_j = x[col]
    pos_j = pos[col]
    pos_i = centers[row]
    diff = pos_j - pos_i
    dist = jnp.sqrt(jnp.sum(diff ** 2, axis=-1, keepdims=True) + 1e-12)
    feat = jnp.concatenate([x_j, dist], axis=-1)
    edge_emb = jax.nn.relu(feat @ W_msg + b_msg)
    w = edge_emb @ W_pos + b_pos
    pos_msg = diff * w
    edge_emb = edge_emb * vmask[:, None]
    pos_msg = pos_msg * vmask[:, None]
    aggr_x = jax.ops.segment_sum(edge_emb, row, num_segments=M)
    cnt = jax.ops.segment_sum(vmask, row, num_segments=M)
    aggr_pos = jax.ops.segment_sum(pos_msg, row, num_segments=M) / jnp.maximum(cnt, 1.0)[:, None]
    x_dest = x[idx]
    x_out = jax.nn.relu(jnp.concatenate([x_dest, aggr_x], axis=-1) @ W_upd + b_upd)
    pos_out = centers + aggr_pos
    batch_out = batch[idx]
    return (x_out, pos_out, batch_out)


# speedup vs baseline: 1.4960x; 1.4960x over previous
import functools

import jax
import jax.numpy as jnp
from jax.experimental import pallas as pl
from jax.experimental.pallas import tpu as pltpu
from jax.experimental.pallas import tpu_sc as plsc

N = 10000
D = 128
HID = 128
OUT = 256
RATIO = 0.5
R = 0.2
MAX_NB = 32
M = int(N * RATIO)

SUB = 8
LANES = -(-N // (SUB * 128)) * 128
TOTAL = SUB * LANES


def _fps_body(posr_ref, xg_ref, yg_ref, zg_ref, idx_ref, cen_ref):
    X = xg_ref[:, :]
    Y = yg_ref[:, :]
    Z = zg_ref[:, :]
    sub_i = jax.lax.broadcasted_iota(jnp.int32, (SUB, LANES), 0)
    lane_i = jax.lax.broadcasted_iota(jnp.int32, (SUB, LANES), 1)
    iota = sub_i * LANES + lane_i
    valid = iota < N
    big = jnp.int32(N)
    dd0 = jnp.where(valid, jnp.inf, -jnp.inf).astype(jnp.float32)

    idx_ref[pl.ds(0, 1), :] = jnp.zeros((1, 1), jnp.int32)
    cen_ref[pl.ds(0, 1), :] = posr_ref[pl.ds(0, 1), :]

    def body(i, carry):
        dd, last = carry
        rowp = posr_ref[pl.ds(last, 1), :]
        bx = jnp.broadcast_to(rowp[0:1, 0:1], (SUB, LANES))
        by = jnp.broadcast_to(rowp[0:1, 1:2], (SUB, LANES))
        bz = jnp.broadcast_to(rowp[0:1, 2:3], (SUB, LANES))
        dx = X - bx
        dy = Y - by
        dz = Z - bz
        d = dx * dx + dy * dy + dz * dz
        dd = jnp.minimum(dd, d)
        mx = jnp.max(dd)
        nxt = jnp.min(jnp.where(dd == mx, iota, big)).astype(jnp.int32)
        idx_ref[pl.ds(i, 1), :] = jnp.full((1, 1), 0, jnp.int32) + nxt
        cen_ref[pl.ds(i, 1), :] = posr_ref[pl.ds(nxt, 1), :]
        return (dd, nxt)

    jax.lax.fori_loop(1, M, body, (dd0, jnp.int32(0)))


def _fps_pallas(pos):
    pad = jnp.zeros((TOTAL - N,), jnp.float32)
    xg = jnp.concatenate([pos[:, 0], pad]).reshape(SUB, LANES)
    yg = jnp.concatenate([pos[:, 1], pad]).reshape(SUB, LANES)
    zg = jnp.concatenate([pos[:, 2], pad]).reshape(SUB, LANES)
    idx2, cen = pl.pallas_call(
        _fps_body,
        out_shape=(
            jax.ShapeDtypeStruct((M, 1), jnp.int32),
            jax.ShapeDtypeStruct((M, 3), jnp.float32),
        ),
    )(pos, xg, yg, zg)
    return idx2[:, 0], cen


QBITS = 17
JBITS = 14
JMASK = (1 << JBITS) - 1
KSCALE = float((2 ** QBITS - 1) / (R * R))
SENT = 2 ** 31 - 1
PPAD = 10240
MPAD = 5120
CBLK = 256
PBLK = 2560


def _keys_body(cen_ref, posT_ref, pn2_ref, keys_ref):
    c = cen_ref[:, :]
    pT = posT_ref[:, :]
    pn2 = pn2_ref[:, :]
    cn2 = jnp.sum(c * c, axis=1, keepdims=True)
    dot = jnp.dot(c, pT, preferred_element_type=jnp.float32)
    d2 = cn2 + pn2 - 2.0 * dot
    q = jnp.maximum(jnp.int32(0), (d2 * KSCALE).astype(jnp.int32))
    j = jax.lax.broadcasted_iota(jnp.int32, (CBLK, PBLK), 1)
    base = pl.program_id(1) * PBLK
    key = jnp.where(d2 <= R * R, (q << JBITS) | (j + base), jnp.int32(SENT))
    keys_ref[:, :] = key


def _keys_pallas(centers, pos):
    cenp = jnp.concatenate(
        [centers, jnp.full((MPAD - M, 3), 100.0, jnp.float32)], axis=0)
    posT = jnp.concatenate(
        [pos.T, jnp.full((3, PPAD - N), 50.0, jnp.float32)], axis=1)
    pn2 = jnp.sum(posT * posT, axis=0, keepdims=True)
    keys = pl.pallas_call(
        _keys_body,
        grid=(MPAD // CBLK, PPAD // PBLK),
        in_specs=[
            pl.BlockSpec((CBLK, 3), lambda i, j: (i, 0)),
            pl.BlockSpec((3, PBLK), lambda i, j: (0, j)),
            pl.BlockSpec((1, PBLK), lambda i, j: (0, j)),
        ],
        out_specs=pl.BlockSpec((CBLK, PBLK), lambda i, j: (i, j)),
        out_shape=jax.ShapeDtypeStruct((MPAD, PPAD), jnp.int32),
    )(cenp, posT, pn2)
    return keys


def _select_topk(keys):
    skey = -jax.lax.top_k(-keys, MAX_NB)[0]
    valid = skey != SENT
    nbr = jnp.where(valid, skey & JMASK, 0)
    return nbr, valid


TBLC = 256
YBLK = 1000


def _table_body(x_ref, pos_ref, wm_ref, bm_ref, tab_ref):
    xb = x_ref[:, :]
    y = jnp.dot(xb, wm_ref[:, :], preferred_element_type=jnp.float32) + bm_ref[:, :]
    tab_ref[:, 0:D] = y
    tab_ref[:, D:D + 3] = pos_ref[:, :]
    tab_ref[:, D + 3:TBLC] = jnp.zeros((YBLK, TBLC - D - 3), jnp.float32)


def _table_pallas(x, pos, W_msg, b_msg):
    tab = pl.pallas_call(
        _table_body,
        grid=(N // YBLK,),
        in_specs=[
            pl.BlockSpec((YBLK, D), lambda i: (i, 0)),
            pl.BlockSpec((YBLK, 3), lambda i: (i, 0)),
            pl.BlockSpec((D, HID), lambda i: (0, 0)),
            pl.BlockSpec((1, HID), lambda i: (0, 0)),
        ],
        out_specs=pl.BlockSpec((YBLK, TBLC), lambda i: (i, 0)),
        out_shape=jax.ShapeDtypeStruct((N, TBLC), jnp.float32),
    )(x, pos, W_msg[:D], b_msg[None, :])
    return tab


SC_NW = 32
EDGES = MAX_NB * MPAD
E_PER_W = EDGES // SC_NW
ECHUNK = 128
NECH = E_PER_W // ECHUNK
C_PER_W = MPAD // SC_NW


def _sc_gather_body(tab_hbm, x_hbm, bat_hbm, colt_hbm, idx_hbm,
                    ge_hbm, gx_hbm, gb_hbm,
                    idx_v, rows_v, xrows_v, brows_v, sem):
    c = jax.lax.axis_index("c")
    s = jax.lax.axis_index("s")
    wid = s * 2 + c
    ebase = wid * E_PER_W

    def echunk(t, carry):
        off = ebase + t * ECHUNK
        pltpu.sync_copy(colt_hbm.at[pl.ds(off, ECHUNK)], idx_v)
        pltpu.async_copy(tab_hbm.at[idx_v], rows_v, sem).wait()
        pltpu.sync_copy(rows_v, ge_hbm.at[pl.ds(off, ECHUNK)])
        return carry

    jax.lax.fori_loop(0, NECH, echunk, 0)

    cbase = wid * C_PER_W
    pltpu.sync_copy(idx_hbm.at[pl.ds(cbase, 128)], idx_v)
    pltpu.async_copy(x_hbm.at[idx_v], xrows_v, sem).wait()
    pltpu.sync_copy(xrows_v, gx_hbm.at[pl.ds(cbase, 128)])
    pltpu.async_copy(bat_hbm.at[idx_v], brows_v, sem).wait()
    pltpu.sync_copy(brows_v, gb_hbm.at[pl.ds(cbase, 128)])

    idx_v2 = idx_v.at[pl.ds(0, 32)]
    pltpu.sync_copy(idx_hbm.at[pl.ds(cbase + 128, 32)], idx_v2)
    xrows_v2 = xrows_v.at[pl.ds(0, 32)]
    pltpu.async_copy(x_hbm.at[idx_v2], xrows_v2, sem).wait()
    pltpu.sync_copy(xrows_v2, gx_hbm.at[pl.ds(cbase + 128, 32)])
    brows_v2 = brows_v.at[pl.ds(0, 32)]
    pltpu.async_copy(bat_hbm.at[idx_v2], brows_v2, sem).wait()
    pltpu.sync_copy(brows_v2, gb_hbm.at[pl.ds(cbase + 128, 32)])


def _sc_gather(tab, x, batpad, col_t, idx_pad):
    mesh = plsc.VectorSubcoreMesh(core_axis_name="c", subcore_axis_name="s")
    fn = functools.partial(
        pl.kernel,
        mesh=mesh,
        out_type=(
            jax.ShapeDtypeStruct((EDGES, TBLC), jnp.float32),
            jax.ShapeDtypeStruct((MPAD, D), jnp.float32),
            jax.ShapeDtypeStruct((MPAD, 128), jnp.int32),
        ),
        scratch_types=[
            pltpu.VMEM((ECHUNK,), jnp.int32),
            pltpu.VMEM((ECHUNK, TBLC), jnp.float32),
            pltpu.VMEM((128, D), jnp.float32),
            pltpu.VMEM((128, 128), jnp.int32),
            pltpu.SemaphoreType.DMA,
        ],
    )(_sc_gather_body)
    return fn(tab, x, batpad, col_t, idx_pad)


EBLK = 128


def _edge_body(ge_ref, vm_ref, cen_ref, gx_ref, wl_ref, wp_ref, bp_ref,
               bu_ref, wua_ref, wub_ref, xo_ref, po_ref):
    wlast = wl_ref[:, :]
    cen = cen_ref[:, :]
    cnt = jnp.sum(vm_ref[:, :], axis=1, keepdims=True)

    def kstep(k, carry):
        acc_x, acc_p = carry
        yk = ge_ref[k, :, 0:D]
        pj = ge_ref[k, :, D:D + 3]
        diff = pj - cen
        d2e = jnp.sum(diff * diff, axis=1, keepdims=True)
        dist = jnp.sqrt(d2e + 1e-12)
        e = jax.nn.relu(yk + dist * wlast)
        w3 = jnp.dot(e, wp_ref[:, :], preferred_element_type=jnp.float32) + bp_ref[:, :]
        vm = jnp.clip(cnt - k.astype(jnp.float32), 0.0, 1.0)
        acc_x = acc_x + e * vm
        acc_p = acc_p + diff * w3 * vm
        return (acc_x, acc_p)

    acc_x0 = jnp.zeros((EBLK, HID), jnp.float32)
    acc_p0 = jnp.zeros((EBLK, 3), jnp.float32)
    acc_x, acc_p = jax.lax.fori_loop(0, MAX_NB, kstep, (acc_x0, acc_p0))
    xo = (jnp.dot(gx_ref[:, :], wua_ref[:, :], preferred_element_type=jnp.float32)
          + jnp.dot(acc_x, wub_ref[:, :], preferred_element_type=jnp.float32)
          + bu_ref[:, :])
    xo_ref[:, :] = jax.nn.relu(xo)
    po_ref[:, :] = cen + acc_p / jnp.maximum(cnt, 1.0)


def _edge_pallas(ge, vmask, centers_pad, gx, W_msg, W_pos, b_pos, b_upd, W_upd):
    ge3 = ge.reshape(MAX_NB, MPAD, TBLC)
    xo, po = pl.pallas_call(
        _edge_body,
        grid=(MPAD // EBLK,),
        in_specs=[
            pl.BlockSpec((MAX_NB, EBLK, TBLC), lambda i: (0, i, 0)),
            pl.BlockSpec((EBLK, MAX_NB), lambda i: (i, 0)),
            pl.BlockSpec((EBLK, 3), lambda i: (i, 0)),
            pl.BlockSpec((EBLK, D), lambda i: (i, 0)),
            pl.BlockSpec((1, HID), lambda i: (0, 0)),
            pl.BlockSpec((HID, 3), lambda i: (0, 0)),
            pl.BlockSpec((1, 3), lambda i: (0, 0)),
            pl.BlockSpec((1, OUT), lambda i: (0, 0)),
            pl.BlockSpec((D, OUT), lambda i: (0, 0)),
            pl.BlockSpec((HID, OUT), lambda i: (0, 0)),
        ],
        out_specs=(
            pl.BlockSpec((EBLK, OUT), lambda i: (i, 0)),
            pl.BlockSpec((EBLK, 3), lambda i: (i, 0)),
        ),
        out_shape=(
            jax.ShapeDtypeStruct((MPAD, OUT), jnp.float32),
            jax.ShapeDtypeStruct((MPAD, 3), jnp.float32),
        ),
    )(ge3, vmask, centers_pad, gx, W_msg[D][None, :], W_pos, b_pos[None, :],
      b_upd[None, :], W_upd[:D], W_upd[D:])
    return xo[:M], po[:M]


def kernel(x, pos, W_msg, b_msg, W_pos, b_pos, W_upd, b_upd, batch):
    idx, centers = _fps_pallas(pos)
    keys = _keys_pallas(centers, pos)
    nbr, valid = _select_topk(keys)
    vmask = valid.astype(jnp.float32)
    col_t = nbr.T.reshape(-1)
    tab = _table_pallas(x, pos, W_msg, b_msg)
    batpad = jnp.broadcast_to(batch[:, None], (N, 128)).astype(jnp.int32)
    idx_pad = jnp.concatenate([idx, jnp.zeros((MPAD - M,), jnp.int32)])
    ge, gx, gb = _sc_gather(tab, x, batpad, col_t, idx_pad)
    centers_pad = jnp.concatenate(
        [centers, jnp.zeros((MPAD - M, 3), jnp.float32)], axis=0)
    x_out, pos_out = _edge_pallas(ge, vmask, centers_pad, gx, W_msg, W_pos,
                                  b_pos, b_upd, W_upd)
    batch_out = gb[:M, 0]
    return (x_out, pos_out, batch_out)


# speedup vs baseline: 1.6245x; 1.0859x over previous
import functools

import jax
import jax.numpy as jnp
from jax.experimental import pallas as pl
from jax.experimental.pallas import tpu as pltpu
from jax.experimental.pallas import tpu_sc as plsc

N = 10000
D = 128
HID = 128
OUT = 256
RATIO = 0.5
R = 0.2
MAX_NB = 32
M = int(N * RATIO)

SUB = 8
LANES = -(-N // (SUB * 128)) * 128
TOTAL = SUB * LANES


def _fps_body(posr_ref, xg_ref, yg_ref, zg_ref, idx_ref, cen_ref):
    sub_i = jax.lax.broadcasted_iota(jnp.int32, (SUB, LANES), 0)
    lane_i = jax.lax.broadcasted_iota(jnp.int32, (SUB, LANES), 1)
    iota = sub_i * LANES + lane_i
    valid = iota < N
    big = jnp.int32(N)
    dd0 = jnp.where(valid, jnp.inf, -jnp.inf).astype(jnp.float32)

    idx_ref[pl.ds(0, 1), :] = jnp.zeros((1, 1), jnp.int32)
    cen_ref[pl.ds(0, 1), :] = posr_ref[pl.ds(0, 1), :]

    def body(i, carry):
        dd, last = carry
        rowp = posr_ref[pl.ds(last, 1), :]
        bx = jnp.broadcast_to(rowp[0:1, 0:1], (SUB, LANES))
        by = jnp.broadcast_to(rowp[0:1, 1:2], (SUB, LANES))
        bz = jnp.broadcast_to(rowp[0:1, 2:3], (SUB, LANES))
        dx = xg_ref[:, :] - bx
        dy = yg_ref[:, :] - by
        dz = zg_ref[:, :] - bz
        d = dx * dx + dy * dy + dz * dz
        dd = jnp.minimum(dd, d)
        nxt = jnp.argmax(dd.reshape(1, -1), axis=1)[0].astype(jnp.int32)
        idx_ref[pl.ds(i, 1), :] = jnp.full((1, 1), 0, jnp.int32) + nxt
        cen_ref[pl.ds(i, 1), :] = posr_ref[pl.ds(nxt, 1), :]
        return (dd, nxt)

    jax.lax.fori_loop(1, M, body, (dd0, jnp.int32(0)))


def _fps_pallas(pos):
    pad = jnp.zeros((TOTAL - N,), jnp.float32)
    xg = jnp.concatenate([pos[:, 0], pad]).reshape(SUB, LANES)
    yg = jnp.concatenate([pos[:, 1], pad]).reshape(SUB, LANES)
    zg = jnp.concatenate([pos[:, 2], pad]).reshape(SUB, LANES)
    idx2, cen = pl.pallas_call(
        _fps_body,
        out_shape=(
            jax.ShapeDtypeStruct((M, 1), jnp.int32),
            jax.ShapeDtypeStruct((M, 3), jnp.float32),
        ),
    )(pos, xg, yg, zg)
    return idx2[:, 0], cen


QBITS = 17
JBITS = 14
JMASK = (1 << JBITS) - 1
KSCALE = float((2 ** QBITS - 1) / (R * R))
SENT = 2 ** 31 - 1
PPAD = 10240
MPAD = 5120
CBLK = 256
PBLK = 2560


def _keys_body(cen_ref, posT_ref, pn2_ref, keys_ref):
    c = cen_ref[:, :]
    pT = posT_ref[:, :]
    pn2 = pn2_ref[:, :]
    cn2 = jnp.sum(c * c, axis=1, keepdims=True)
    dot = jnp.dot(c, pT, preferred_element_type=jnp.float32)
    d2 = cn2 + pn2 - 2.0 * dot
    q = jnp.maximum(jnp.int32(0), (d2 * KSCALE).astype(jnp.int32))
    j = jax.lax.broadcasted_iota(jnp.int32, (CBLK, PBLK), 1)
    base = pl.program_id(1) * PBLK
    key = jnp.where(d2 <= R * R, (q << JBITS) | (j + base), jnp.int32(SENT))
    keys_ref[:, :] = key


def _keys_pallas(centers, pos):
    cenp = jnp.concatenate(
        [centers, jnp.full((MPAD - M, 3), 100.0, jnp.float32)], axis=0)
    posT = jnp.concatenate(
        [pos.T, jnp.full((3, PPAD - N), 50.0, jnp.float32)], axis=1)
    pn2 = jnp.sum(posT * posT, axis=0, keepdims=True)
    keys = pl.pallas_call(
        _keys_body,
        grid=(MPAD // CBLK, PPAD // PBLK),
        in_specs=[
            pl.BlockSpec((CBLK, 3), lambda i, j: (i, 0)),
            pl.BlockSpec((3, PBLK), lambda i, j: (0, j)),
            pl.BlockSpec((1, PBLK), lambda i, j: (0, j)),
        ],
        out_specs=pl.BlockSpec((CBLK, PBLK), lambda i, j: (i, j)),
        out_shape=jax.ShapeDtypeStruct((MPAD, PPAD), jnp.int32),
    )(cenp, posT, pn2)
    return keys


def _select_topk(keys):
    skey = -jax.lax.top_k(-keys, MAX_NB)[0]
    valid = skey != SENT
    nbr = jnp.where(valid, skey & JMASK, 0)
    return nbr, valid


TBLC = 256
YBLK = 1000


def _table_body(x_ref, pos_ref, wm_ref, bm_ref, tab_ref):
    xb = x_ref[:, :]
    y = jnp.dot(xb, wm_ref[:, :], preferred_element_type=jnp.float32) + bm_ref[:, :]
    tab_ref[:, 0:D] = y
    tab_ref[:, D:D + 3] = pos_ref[:, :]
    tab_ref[:, D + 3:TBLC] = jnp.zeros((YBLK, TBLC - D - 3), jnp.float32)


def _table_pallas(x, pos, W_msg, b_msg):
    tab = pl.pallas_call(
        _table_body,
        grid=(N // YBLK,),
        in_specs=[
            pl.BlockSpec((YBLK, D), lambda i: (i, 0)),
            pl.BlockSpec((YBLK, 3), lambda i: (i, 0)),
            pl.BlockSpec((D, HID), lambda i: (0, 0)),
            pl.BlockSpec((1, HID), lambda i: (0, 0)),
        ],
        out_specs=pl.BlockSpec((YBLK, TBLC), lambda i: (i, 0)),
        out_shape=jax.ShapeDtypeStruct((N, TBLC), jnp.float32),
    )(x, pos, W_msg[:D], b_msg[None, :])
    return tab


SC_NW = 32
EDGES = MAX_NB * MPAD
E_PER_W = EDGES // SC_NW
ECHUNK = 128
NECH = E_PER_W // ECHUNK
C_PER_W = MPAD // SC_NW


def _sc_gather_body(tab_hbm, x_hbm, bat_hbm, colt_hbm, idx_hbm,
                    ge_hbm, gx_hbm, gb_hbm,
                    idx_v, rows_v, xrows_v, brows_v, sem):
    c = jax.lax.axis_index("c")
    s = jax.lax.axis_index("s")
    wid = s * 2 + c
    ebase = wid * E_PER_W

    def echunk(t, carry):
        off = ebase + t * ECHUNK
        pltpu.sync_copy(colt_hbm.at[pl.ds(off, ECHUNK)], idx_v)
        pltpu.async_copy(tab_hbm.at[idx_v], rows_v, sem).wait()
        pltpu.sync_copy(rows_v, ge_hbm.at[pl.ds(off, ECHUNK)])
        return carry

    jax.lax.fori_loop(0, NECH, echunk, 0)

    cbase = wid * C_PER_W
    pltpu.sync_copy(idx_hbm.at[pl.ds(cbase, 128)], idx_v)
    pltpu.async_copy(x_hbm.at[idx_v], xrows_v, sem).wait()
    pltpu.sync_copy(xrows_v, gx_hbm.at[pl.ds(cbase, 128)])
    pltpu.async_copy(bat_hbm.at[idx_v], brows_v, sem).wait()
    pltpu.sync_copy(brows_v, gb_hbm.at[pl.ds(cbase, 128)])

    idx_v2 = idx_v.at[pl.ds(0, 32)]
    pltpu.sync_copy(idx_hbm.at[pl.ds(cbase + 128, 32)], idx_v2)
    xrows_v2 = xrows_v.at[pl.ds(0, 32)]
    pltpu.async_copy(x_hbm.at[idx_v2], xrows_v2, sem).wait()
    pltpu.sync_copy(xrows_v2, gx_hbm.at[pl.ds(cbase + 128, 32)])
    brows_v2 = brows_v.at[pl.ds(0, 32)]
    pltpu.async_copy(bat_hbm.at[idx_v2], brows_v2, sem).wait()
    pltpu.sync_copy(brows_v2, gb_hbm.at[pl.ds(cbase + 128, 32)])


def _sc_gather(tab, x, batpad, col_t, idx_pad):
    mesh = plsc.VectorSubcoreMesh(core_axis_name="c", subcore_axis_name="s")
    fn = functools.partial(
        pl.kernel,
        mesh=mesh,
        out_type=(
            jax.ShapeDtypeStruct((EDGES, TBLC), jnp.float32),
            jax.ShapeDtypeStruct((MPAD, D), jnp.float32),
            jax.ShapeDtypeStruct((MPAD, 128), jnp.int32),
        ),
        scratch_types=[
            pltpu.VMEM((ECHUNK,), jnp.int32),
            pltpu.VMEM((ECHUNK, TBLC), jnp.float32),
            pltpu.VMEM((128, D), jnp.float32),
            pltpu.VMEM((128, 128), jnp.int32),
            pltpu.SemaphoreType.DMA,
        ],
    )(_sc_gather_body)
    return fn(tab, x, batpad, col_t, idx_pad)


EBLK = 128


def _edge_body(ge_ref, vm_ref, cen_ref, gx_ref, wl_ref, wp_ref, bp_ref,
               bu_ref, wua_ref, wub_ref, xo_ref, po_ref):
    wlast = wl_ref[:, :]
    cen = cen_ref[:, :]
    cnt = jnp.sum(vm_ref[:, :], axis=1, keepdims=True)

    def kstep(k, carry):
        acc_x, acc_p = carry
        yk = ge_ref[k, :, 0:D]
        pj = ge_ref[k, :, D:D + 3]
        diff = pj - cen
        d2e = jnp.sum(diff * diff, axis=1, keepdims=True)
        dist = jnp.sqrt(d2e + 1e-12)
        e = jax.nn.relu(yk + dist * wlast)
        w3 = jnp.dot(e, wp_ref[:, :], preferred_element_type=jnp.float32) + bp_ref[:, :]
        vm = jnp.clip(cnt - k.astype(jnp.float32), 0.0, 1.0)
        acc_x = acc_x + e * vm
        acc_p = acc_p + diff * w3 * vm
        return (acc_x, acc_p)

    acc_x0 = jnp.zeros((EBLK, HID), jnp.float32)
    acc_p0 = jnp.zeros((EBLK, 3), jnp.float32)
    acc_x, acc_p = jax.lax.fori_loop(0, MAX_NB, kstep, (acc_x0, acc_p0))
    xo = (jnp.dot(gx_ref[:, :], wua_ref[:, :], preferred_element_type=jnp.float32)
          + jnp.dot(acc_x, wub_ref[:, :], preferred_element_type=jnp.float32)
          + bu_ref[:, :])
    xo_ref[:, :] = jax.nn.relu(xo)
    po_ref[:, :] = cen + acc_p / jnp.maximum(cnt, 1.0)


def _edge_pallas(ge, vmask, centers_pad, gx, W_msg, W_pos, b_pos, b_upd, W_upd):
    ge3 = ge.reshape(MAX_NB, MPAD, TBLC)
    xo, po = pl.pallas_call(
        _edge_body,
        grid=(MPAD // EBLK,),
        in_specs=[
            pl.BlockSpec((MAX_NB, EBLK, TBLC), lambda i: (0, i, 0)),
            pl.BlockSpec((EBLK, MAX_NB), lambda i: (i, 0)),
            pl.BlockSpec((EBLK, 3), lambda i: (i, 0)),
            pl.BlockSpec((EBLK, D), lambda i: (i, 0)),
            pl.BlockSpec((1, HID), lambda i: (0, 0)),
            pl.BlockSpec((HID, 3), lambda i: (0, 0)),
            pl.BlockSpec((1, 3), lambda i: (0, 0)),
            pl.BlockSpec((1, OUT), lambda i: (0, 0)),
            pl.BlockSpec((D, OUT), lambda i: (0, 0)),
            pl.BlockSpec((HID, OUT), lambda i: (0, 0)),
        ],
        out_specs=(
            pl.BlockSpec((EBLK, OUT), lambda i: (i, 0)),
            pl.BlockSpec((EBLK, 3), lambda i: (i, 0)),
        ),
        out_shape=(
            jax.ShapeDtypeStruct((MPAD, OUT), jnp.float32),
            jax.ShapeDtypeStruct((MPAD, 3), jnp.float32),
        ),
    )(ge3, vmask, centers_pad, gx, W_msg[D][None, :], W_pos, b_pos[None, :],
      b_upd[None, :], W_upd[:D], W_upd[D:])
    return xo[:M], po[:M]


def kernel(x, pos, W_msg, b_msg, W_pos, b_pos, W_upd, b_upd, batch):
    idx, centers = _fps_pallas(pos)
    keys = _keys_pallas(centers, pos)
    nbr, valid = _select_topk(keys)
    vmask = valid.astype(jnp.float32)
    col_t = nbr.T.reshape(-1)
    tab = _table_pallas(x, pos, W_msg, b_msg)
    batpad = jnp.broadcast_to(batch[:, None], (N, 128)).astype(jnp.int32)
    idx_pad = jnp.concatenate([idx, jnp.zeros((MPAD - M,), jnp.int32)])
    ge, gx, gb = _sc_gather(tab, x, batpad, col_t, idx_pad)
    centers_pad = jnp.concatenate(
        [centers, jnp.zeros((MPAD - M, 3), jnp.float32)], axis=0)
    x_out, pos_out = _edge_pallas(ge, vmask, centers_pad, gx, W_msg, W_pos,
                                  b_pos, b_upd, W_upd)
    batch_out = gb[:M, 0]
    return (x_out, pos_out, batch_out)
